# Initial kernel scaffold; baseline (speedup 1.0000x reference)
import jax
import jax.numpy as jnp
from jax.experimental import pallas as pl


def kernel(nodes, edges, edge_weights, W1, b1, g1, be1, W2, b2, g2, be2, W3, b3, g3, be3, W4, b4, g4, be4):
    raise NotImplementedError("write your pallas kernel here")


# speedup vs baseline: 14.0908x; 14.0908x over previous
import functools

import jax
import jax.numpy as jnp
from jax import lax
from jax.experimental import pallas as pl
from jax.experimental.pallas import tpu as pltpu
from jax.experimental.pallas import tpu_sc as plsc

NC, NS, L = 2, 16, 16
CHUNK = 128


def _round_up(x, m):
    return -(-x // m) * m


def _make_sc_agg(B, N, D, n_chunks):
    npad = _round_up(N + 1, NS * 8)
    zrows = npad // NS
    e_tile = n_chunks * CHUNK
    mesh = plsc.VectorSubcoreMesh(core_axis_name="c", subcore_axis_name="s")

    @functools.partial(
        pl.kernel,
        out_type=jax.ShapeDtypeStruct((NC, B, npad, D), jnp.float32),
        mesh=mesh,
        compiler_params=pltpu.CompilerParams(needs_layout_passes=False),
        scratch_types=[
            pltpu.VMEM_SHARED((npad, D), jnp.float32),
            pltpu.VMEM((CHUNK, D), jnp.float32),
            pltpu.VMEM((CHUNK,), jnp.int32),
            pltpu.VMEM((CHUNK,), jnp.int32),
            pltpu.VMEM((CHUNK,), jnp.float32),
            pltpu.SemaphoreType.DMA,
        ],
    )
    def sc_agg(nodes_hbm, gidx_hbm, sidx_hbm, w_hbm, zeros_hbm, out_hbm,
               acc, rows, gi, si, wv, sem):
        c = lax.axis_index("c")
        s = lax.axis_index("s")

        for b in range(B):
            pltpu.sync_copy(zeros_hbm, acc.at[pl.ds(s * zrows, zrows)])
            plsc.subcore_barrier()

            def chunk_body(i, carry):
                base = s * e_tile + i * CHUNK
                pltpu.sync_copy(gidx_hbm.at[c, b, pl.ds(base, CHUNK)], gi)
                pltpu.sync_copy(sidx_hbm.at[c, b, pl.ds(base, CHUNK)], si)
                pltpu.sync_copy(w_hbm.at[c, b, pl.ds(base, CHUNK)], wv)
                pltpu.async_copy(nodes_hbm.at[gi], rows, sem).wait()

                def row_body(r, rcarry):
                    wb = plsc.load_gather(
                        wv, [jnp.zeros((L,), jnp.int32) + r])
                    for j in range(D // L):
                        sl = pl.ds(j * L, L)
                        rows[r, sl] = rows[r, sl] * wb
                    return rcarry

                lax.fori_loop(0, CHUNK, row_body, 0)
                pltpu.sync_copy(rows, acc.at[si], add=True)
                return carry

            lax.fori_loop(0, n_chunks, chunk_body, 0)
            plsc.subcore_barrier()

            pltpu.sync_copy(acc.at[pl.ds(s * zrows, zrows)],
                            out_hbm.at[c, b, pl.ds(s * zrows, zrows)])
            plsc.subcore_barrier()

    return sc_agg


def _mlp_body(ai, ao, nd, w1a, w1b, w1c, b1, g1, be1, w2, b2, g2, be2,
              w3, b3, g3, be3, w4, b4, g4, be4, out):
    def ln(x, g, be):
        m = jnp.mean(x, axis=-1, keepdims=True)
        v = jnp.mean((x - m) ** 2, axis=-1, keepdims=True)
        return (x - m) / jnp.sqrt(v + 1e-5) * g + be

    dot = functools.partial(jnp.dot, preferred_element_type=jnp.float32)
    h = (dot(ai[0], w1a[...]) + dot(ao[0], w1b[...]) + dot(nd[0], w1c[...])
         + b1[...])
    h = jnp.tanh(ln(h, g1[...], be1[...]))
    h = jnp.tanh(ln(dot(h, w2[...]) + b2[...], g2[...], be2[...]))
    h = jnp.tanh(ln(dot(h, w3[...]) + b3[...], g3[...], be3[...]))
    h = jnp.tanh(ln(dot(h, w4[...]) + b4[...], g4[...], be4[...]))
    out[0] = h


def _mlp(agg_in, agg_out, nodes, params, row_block):
    B, N, D = nodes.shape
    grid = (B, N // row_block)
    node_spec = pl.BlockSpec((1, row_block, D), lambda b, i: (b, i, 0))
    w_spec = pl.BlockSpec((D, D), lambda b, i: (0, 0))
    v_spec = pl.BlockSpec((1, D), lambda b, i: (0, 0))
    specs = [node_spec] * 3 + [w_spec] * 3 + [v_spec] * 3 + \
        ([w_spec] + [v_spec] * 3) * 3
    return pl.pallas_call(
        _mlp_body,
        grid=grid,
        in_specs=specs,
        out_specs=node_spec,
        out_shape=jax.ShapeDtypeStruct((B, N, D), jnp.float32),
    )(agg_in, agg_out, nodes, *params)


def kernel(nodes, edges, edge_weights, W1, b1, g1, be1, W2, b2, g2, be2,
           W3, b3, g3, be3, W4, b4, g4, be4):
    B, N, D = nodes.shape
    E = edges.shape[1]
    n_chunks = -(-E // (NS * CHUNK))
    e_pad = NS * n_chunks * CHUNK
    pad = e_pad - E

    src = edges[..., 0]
    dst = edges[..., 1]
    offs = (jnp.arange(B, dtype=jnp.int32) * N)[:, None]
    gidx = jnp.stack([src + offs, dst + offs])
    sidx = jnp.stack([dst, src])
    gidx = jnp.pad(gidx, ((0, 0), (0, 0), (0, pad)))
    sidx = jnp.pad(sidx, ((0, 0), (0, 0), (0, pad)), constant_values=N)
    w = jnp.broadcast_to(edge_weights, (NC, B, E))
    w = jnp.pad(w, ((0, 0), (0, 0), (0, pad)))
    nodes_flat = nodes.reshape(B * N, D)
    zeros = jnp.zeros((_round_up(N + 1, NS * 8) // NS, D), jnp.float32)

    agg = _make_sc_agg(B, N, D, n_chunks)(nodes_flat, gidx, sidx, w, zeros)

    params = (W1[:D], W1[D:2 * D], W1[2 * D:],
              b1[None], g1[None], be1[None],
              W2, b2[None], g2[None], be2[None],
              W3, b3[None], g3[None], be3[None],
              W4, b4[None], g4[None], be4[None])
    return _mlp(agg[0], agg[1], nodes, params, row_block=400)
